# baseline (speedup 1.0000x reference)
import functools

import jax
import jax.numpy as jnp
from jax import lax
from jax.experimental import pallas as pl
from jax.experimental.pallas import tpu as pltpu
from jax.experimental.pallas import tpu_sc as plsc

T = 2048
HIDDEN = 768
DFF = 512
N_ROUTED = 8
N_TOTAL = 72
NPAD = 128
SCALE = 2.5
NEG = -1e30

TILE = 128
NTILES = 40
PADROWS = NTILES * TILE
NTP = 48
NW1 = 16
TPW1 = T // NW1
NW2 = 32
TPW2 = T // NW2
SRCN = PADROWS + 64
GCHUNK = 64
NCHUNK = PADROWS // GCHUNK


def _router_body(x_ref, rw_ref, bias_ref, e1_ref, e2_ref, w1_ref, w2_ref, zw_ref):
    x = x_ref[...]
    logits = lax.dot_general(x, rw_ref[...], (((1,), (1,)), ((), ())),
                             preferred_element_type=jnp.float32)
    col = lax.broadcasted_iota(jnp.int32, (T, NPAD), 1)
    valid = col < N_TOTAL
    logits = jnp.where(valid, logits, NEG)
    m = jnp.max(logits, axis=1, keepdims=True)
    p = jnp.exp(logits - m)
    p = jnp.where(valid, p, 0.0)
    scores = p / jnp.sum(p, axis=1, keepdims=True)
    sfc = jnp.where(valid, scores + bias_ref[...], NEG)

    m1 = jnp.max(sfc, axis=1, keepdims=True)
    i1 = jnp.min(jnp.where(sfc == m1, col, NPAD), axis=1, keepdims=True)
    sfc2 = jnp.where(col == i1, NEG, sfc)
    m2 = jnp.max(sfc2, axis=1, keepdims=True)
    i2 = jnp.min(jnp.where(sfc2 == m2, col, NPAD), axis=1, keepdims=True)

    s1 = jnp.sum(jnp.where(col == i1, scores, 0.0), axis=1, keepdims=True) * SCALE
    s2 = jnp.sum(jnp.where(col == i2, scores, 0.0), axis=1, keepdims=True) * SCALE

    z1 = i1 >= N_ROUTED
    z2 = i2 >= N_ROUTED
    e1_ref[...] = jnp.where(z1, -1, i1)
    e2_ref[...] = jnp.where(z2, -1, i2)
    w1_ref[...] = jnp.broadcast_to(jnp.where(z1, 0.0, s1), (T, 16))
    w2_ref[...] = jnp.broadcast_to(jnp.where(z2, 0.0, s2), (T, 16))
    zw_ref[...] = jnp.broadcast_to(
        jnp.where(z1, s1, 0.0) + jnp.where(z2, s2, 0.0), (T, 16))


def _router(x, rw_pad, bias_pad):
    v = jax.ShapeDtypeStruct((T, 16), jnp.float32)
    iv = jax.ShapeDtypeStruct((T, 1), jnp.int32)
    return pl.pallas_call(_router_body, out_shape=(iv, iv, v, v, v))(
        x, rw_pad, bias_pad)


def _dispatch_body(e1_hbm, e2_hbm, x_hbm,
                   pos1_hbm, pos2_hbm, te_hbm, xt_hbm, na_hbm, xd_hbm,
                   e1_v, e2_v, cnt_v, hist_v, pos_v, tok_v, idx_v, zero_v,
                   src_v, meta_v, meta2_v, rows_v, hist_sh, src_sh, sem):
    wid = lax.axis_index("s")
    iota = lax.broadcasted_iota(jnp.int32, (16,), 0)
    zeros = jnp.zeros((16,), jnp.int32)

    for j in range(PADROWS // NW1 // 16):
        zero_v[pl.ds(j * 16, 16)] = zeros
    pltpu.sync_copy(zero_v, src_sh.at[pl.ds(wid * (PADROWS // NW1), PADROWS // NW1)])

    pltpu.sync_copy(e1_hbm.at[pl.ds(wid * TPW1, TPW1)], e1_v)
    pltpu.sync_copy(e2_hbm.at[pl.ds(wid * TPW1, TPW1)], e2_v)
    counts = zeros
    for src in (e1_v, e2_v):
        for j in range(TPW1 // 16):
            v = src[pl.ds(j * 16, 16)]
            for e in range(N_ROUTED):
                c = jnp.sum((v == e).astype(jnp.int32))
                counts = counts + jnp.where(iota == e, c, 0)
    cnt_v[...] = counts
    pltpu.sync_copy(cnt_v, hist_sh.at[pl.ds(wid * 16, 16)])
    plsc.subcore_barrier()

    pltpu.sync_copy(hist_sh, hist_v)
    base = zeros
    tot = zeros
    for w in range(NW1):
        row = hist_v[pl.ds(w * 16, 16)]
        tot = tot + row
        base = base + jnp.where(w < wid, row, zeros)
    padded = ((tot + (TILE - 1)) // TILE) * TILE
    offs_inc = plsc.cumsum(padded)
    offs_exc = offs_inc - padded
    start = offs_exc + base
    total_rows = jnp.max(offs_inc)
    ntact = total_rows // TILE

    running = start
    for si, src in enumerate((e1_v, e2_v)):
        for j in range(TPW1 // 16):
            v = src[pl.ds(j * 16, 16)]
            posv = zeros
            addc = zeros
            for e in range(N_ROUTED):
                mk = v == e
                mi = mk.astype(jnp.int32)
                prefix = plsc.cumsum(mi) - mi
                st_e = jnp.sum(jnp.where(iota == e, running, 0))
                posv = jnp.where(mk, st_e + prefix, posv)
                addc = addc + jnp.where(iota == e, jnp.sum(mi), 0)
            running = running + addc
            pos_v[pl.ds(si * TPW1 + j * 16, 16)] = posv
            tok_v[si, pl.ds(j * 16, 16)] = wid * TPW1 + j * 16 + iota
            idx_v[si, pl.ds(j * 16, 16)] = jnp.where(
                v >= 0, posv, PADROWS + wid)
    for si in range(2):
        pltpu.sync_copy(tok_v.at[si], src_sh.at[idx_v.at[si]])
    pltpu.sync_copy(pos_v.at[pl.ds(0, TPW1)], pos1_hbm.at[pl.ds(wid * TPW1, TPW1)])
    pltpu.sync_copy(pos_v.at[pl.ds(TPW1, TPW1)], pos2_hbm.at[pl.ds(wid * TPW1, TPW1)])

    los = []
    his = []
    for e in range(N_ROUTED):
        los.append(jnp.sum(jnp.where(iota == e, offs_exc, 0)) // TILE)
        his.append(jnp.sum(jnp.where(iota == e, offs_inc, 0)) // TILE)
    lt = ntact - 1
    last_e = jnp.int32(0)
    for e in range(N_ROUTED):
        last_e = jnp.where((lt >= los[e]) & (lt < his[e]), e, last_e)

    @pl.when(wid == 0)
    def _():
        for j in range(NTP // 16):
            tv = iota + j * 16
            texp = zeros + last_e
            for e in range(N_ROUTED):
                texp = jnp.where((tv >= los[e]) & (tv < his[e]), e, texp)
            meta_v[pl.ds(j * 16, 16)] = texp
            meta2_v[pl.ds(j * 16, 16)] = jnp.minimum(
                tv, jnp.maximum(ntact - 1, 0))
        pltpu.sync_copy(meta_v, te_hbm)
        pltpu.sync_copy(meta2_v, xt_hbm)
        cnt_v[...] = zeros + ntact
        pltpu.sync_copy(cnt_v, na_hbm)

    plsc.subcore_barrier()
    for j in range(NCHUNK // NW1):
        c = wid + j * NW1

        @pl.when(c * GCHUNK < total_rows)
        def _(c=c):
            pltpu.sync_copy(src_sh.at[pl.ds(c * GCHUNK, GCHUNK)], src_v)
            pltpu.async_copy(x_hbm.at[src_v], rows_v, sem).wait()
            pltpu.sync_copy(rows_v, xd_hbm.at[pl.ds(c * GCHUNK, GCHUNK)])


def _dispatch(e1, e2, x):
    mesh = plsc.VectorSubcoreMesh(core_axis_name="c", subcore_axis_name="s",
                                  num_cores=1, num_subcores=NW1)
    f = pl.kernel(
        _dispatch_body,
        out_type=[
            jax.ShapeDtypeStruct((T,), jnp.int32),
            jax.ShapeDtypeStruct((T,), jnp.int32),
            jax.ShapeDtypeStruct((NTP,), jnp.int32),
            jax.ShapeDtypeStruct((NTP,), jnp.int32),
            jax.ShapeDtypeStruct((16,), jnp.int32),
            jax.ShapeDtypeStruct((PADROWS, HIDDEN), jnp.float32),
        ],
        mesh=mesh,
        compiler_params=pltpu.CompilerParams(needs_layout_passes=False),
        scratch_types=[
            pltpu.VMEM((TPW1,), jnp.int32),
            pltpu.VMEM((TPW1,), jnp.int32),
            pltpu.VMEM((16,), jnp.int32),
            pltpu.VMEM((NW1 * 16,), jnp.int32),
            pltpu.VMEM((TPW1 * 2,), jnp.int32),
            pltpu.VMEM((2, TPW1), jnp.int32),
            pltpu.VMEM((2, TPW1), jnp.int32),
            pltpu.VMEM((PADROWS // NW1,), jnp.int32),
            pltpu.VMEM((GCHUNK,), jnp.int32),
            pltpu.VMEM((NTP,), jnp.int32),
            pltpu.VMEM((NTP,), jnp.int32),
            pltpu.VMEM((GCHUNK, HIDDEN), jnp.float32),
            pltpu.VMEM_SHARED((NW1 * 16,), jnp.int32),
            pltpu.VMEM_SHARED((SRCN,), jnp.int32),
            pltpu.SemaphoreType.DMA,
        ],
    )
    return f(e1, e2, x)


def _gemm_body(te_ref, xt_ref, na_ref, xd_ref, wg_ref, wu_ref, wd_ref, y_ref):
    t = pl.program_id(0)

    @pl.when(t < na_ref[0])
    def _():
        x = xd_ref[...]
        g = jnp.dot(x, wg_ref[0], preferred_element_type=jnp.float32)
        u = jnp.dot(x, wu_ref[0], preferred_element_type=jnp.float32)
        h = g * lax.logistic(g) * u
        y_ref[...] = jnp.dot(h, wd_ref[0], preferred_element_type=jnp.float32)


def _gemm(te, xt, na, xdisp, w_gate, w_up, w_down):
    grid_spec = pltpu.PrefetchScalarGridSpec(
        num_scalar_prefetch=3,
        grid=(NTILES,),
        in_specs=[
            pl.BlockSpec((TILE, HIDDEN), lambda t, te, xt, na: (xt[t], 0)),
            pl.BlockSpec((1, HIDDEN, DFF), lambda t, te, xt, na: (te[t], 0, 0)),
            pl.BlockSpec((1, HIDDEN, DFF), lambda t, te, xt, na: (te[t], 0, 0)),
            pl.BlockSpec((1, DFF, HIDDEN), lambda t, te, xt, na: (te[t], 0, 0)),
        ],
        out_specs=pl.BlockSpec((TILE, HIDDEN), lambda t, te, xt, na: (t, 0)),
    )
    return pl.pallas_call(
        _gemm_body,
        grid_spec=grid_spec,
        out_shape=jax.ShapeDtypeStruct((PADROWS, HIDDEN), jnp.float32),
    )(te, xt, na, xdisp, w_gate, w_up, w_down)


def _combine_body(x_hbm, y_hbm, p1_hbm, p2_hbm, w1_hbm, w2_hbm, zw_hbm,
                  out_hbm, p_v, w_v, x0_v, x1_v, y10_v, y11_v, y20_v, y21_v,
                  o0_v, o1_v, sem0, sem1):
    wid = lax.axis_index("s") * 2 + lax.axis_index("c")
    base = wid * TPW2
    wn = TPW2 * 16
    pltpu.sync_copy(p1_hbm.at[pl.ds(base, TPW2)], p_v.at[pl.ds(0, TPW2)])
    pltpu.sync_copy(p2_hbm.at[pl.ds(base, TPW2)], p_v.at[pl.ds(TPW2, TPW2)])
    pltpu.sync_copy(w1_hbm.at[pl.ds(base * 16, wn)], w_v.at[pl.ds(0, wn)])
    pltpu.sync_copy(w2_hbm.at[pl.ds(base * 16, wn)], w_v.at[pl.ds(wn, wn)])
    pltpu.sync_copy(zw_hbm.at[pl.ds(base * 16, wn)], w_v.at[pl.ds(2 * wn, wn)])
    xb = (x0_v, x1_v)
    y1b = (y10_v, y11_v)
    y2b = (y20_v, y21_v)
    ob = (o0_v, o1_v)
    sems = (sem0, sem1)

    def issue(c):
        b = c % 2
        t0 = base + c * 16
        return (
            pltpu.async_copy(x_hbm.at[pl.ds(t0, 16)], xb[b], sems[b]),
            pltpu.async_copy(
                y_hbm.at[p_v.at[pl.ds(c * 16, 16)]], y1b[b], sems[b]),
            pltpu.async_copy(
                y_hbm.at[p_v.at[pl.ds(TPW2 + c * 16, 16)]], y2b[b], sems[b]),
        )

    nc = TPW2 // 16
    pend = {0: issue(0)}
    for c in range(nc):
        b = c % 2
        for cp in pend.pop(c):
            cp.wait()
        if c + 1 < nc:
            pend[c + 1] = issue(c + 1)
        x_v, y1_v, y2_v, o_v = xb[b], y1b[b], y2b[b], ob[b]

        def tok_body(l, _, x_v=x_v, y1_v=y1_v, y2_v=y2_v, o_v=o_v, c=c):
            w1r = w_v[pl.ds(c * 256 + l * 16, 16)]
            w2r = w_v[pl.ds(wn + c * 256 + l * 16, 16)]
            zwr = w_v[pl.ds(2 * wn + c * 256 + l * 16, 16)]
            for f in range(HIDDEN // 16):
                xr = x_v[l, pl.ds(f * 16, 16)]
                y1r = y1_v[l, pl.ds(f * 16, 16)]
                y2r = y2_v[l, pl.ds(f * 16, 16)]
                o = zwr * xr
                o = o + jnp.where(w1r != 0.0, w1r * y1r, 0.0)
                o = o + jnp.where(w2r != 0.0, w2r * y2r, 0.0)
                o_v[l, pl.ds(f * 16, 16)] = o
            return 0

        lax.fori_loop(0, 16, tok_body, 0)
        pltpu.sync_copy(o_v, out_hbm.at[pl.ds(base + c * 16, 16)])


def _combine(x, y, pos1, pos2, w1, w2, zw):
    mesh = plsc.VectorSubcoreMesh(core_axis_name="c", subcore_axis_name="s",
                                  num_cores=2, num_subcores=16)
    f = pl.kernel(
        _combine_body,
        out_type=jax.ShapeDtypeStruct((T, HIDDEN), jnp.float32),
        mesh=mesh,
        compiler_params=pltpu.CompilerParams(needs_layout_passes=False),
        scratch_types=[
            pltpu.VMEM((2 * TPW2,), jnp.int32),
            pltpu.VMEM((3 * TPW2 * 16,), jnp.float32),
            pltpu.VMEM((16, HIDDEN), jnp.float32),
            pltpu.VMEM((16, HIDDEN), jnp.float32),
            pltpu.VMEM((16, HIDDEN), jnp.float32),
            pltpu.VMEM((16, HIDDEN), jnp.float32),
            pltpu.VMEM((16, HIDDEN), jnp.float32),
            pltpu.VMEM((16, HIDDEN), jnp.float32),
            pltpu.VMEM((16, HIDDEN), jnp.float32),
            pltpu.VMEM((16, HIDDEN), jnp.float32),
            pltpu.SemaphoreType.DMA,
            pltpu.SemaphoreType.DMA,
        ],
    )
    return f(x, y, pos1, pos2, w1, w2, zw)


def kernel(hidden_states, router_weight, e_score_correction_bias, w_gate, w_up, w_down):
    rw_pad = jnp.zeros((NPAD, HIDDEN), jnp.float32).at[:N_TOTAL].set(router_weight)
    bias_pad = jnp.full((1, NPAD), NEG, jnp.float32).at[0, :N_TOTAL].set(
        e_score_correction_bias)
    e1, e2, w1, w2, zw = _router(hidden_states, rw_pad, bias_pad)
    pos1, pos2, te, xt, na, xdisp = _dispatch(
        e1.reshape(T), e2.reshape(T), hidden_states)
    y = _gemm(te, xt, na, xdisp, w_gate, w_up, w_down)
    return _combine(hidden_states, y, pos1, pos2,
                    w1.reshape(T * 16), w2.reshape(T * 16), zw.reshape(T * 16))


# speedup vs baseline: 1.0023x; 1.0023x over previous
import functools

import jax
import jax.numpy as jnp
from jax import lax
from jax.experimental import pallas as pl
from jax.experimental.pallas import tpu as pltpu
from jax.experimental.pallas import tpu_sc as plsc

T = 2048
HIDDEN = 768
DFF = 512
N_ROUTED = 8
N_TOTAL = 72
NPAD = 128
SCALE = 2.5
NEG = -1e30

TILE = 128
NTILES = 40
PADROWS = NTILES * TILE
NTP = 48
NW1 = 16
TPW1 = T // NW1
NW2 = 32
TPW2 = T // NW2
SRCN = PADROWS + 64
GCHUNK = 64
NCHUNK = PADROWS // GCHUNK


def _router_body(x_ref, rw_ref, bias_ref, e1_ref, e2_ref, w1_ref, w2_ref, zw_ref):
    x = x_ref[...]
    logits = lax.dot_general(x, rw_ref[...], (((1,), (1,)), ((), ())),
                             preferred_element_type=jnp.float32)
    col = lax.broadcasted_iota(jnp.int32, (T, NPAD), 1)
    valid = col < N_TOTAL
    logits = jnp.where(valid, logits, NEG)
    m = jnp.max(logits, axis=1, keepdims=True)
    p = jnp.exp(logits - m)
    p = jnp.where(valid, p, 0.0)
    scores = p / jnp.sum(p, axis=1, keepdims=True)
    sfc = jnp.where(valid, scores + bias_ref[...], NEG)

    m1 = jnp.max(sfc, axis=1, keepdims=True)
    i1 = jnp.min(jnp.where(sfc == m1, col, NPAD), axis=1, keepdims=True)
    sfc2 = jnp.where(col == i1, NEG, sfc)
    m2 = jnp.max(sfc2, axis=1, keepdims=True)
    i2 = jnp.min(jnp.where(sfc2 == m2, col, NPAD), axis=1, keepdims=True)

    s1 = jnp.sum(jnp.where(col == i1, scores, 0.0), axis=1, keepdims=True) * SCALE
    s2 = jnp.sum(jnp.where(col == i2, scores, 0.0), axis=1, keepdims=True) * SCALE

    z1 = i1 >= N_ROUTED
    z2 = i2 >= N_ROUTED
    e1_ref[...] = jnp.where(z1, -1, i1)
    e2_ref[...] = jnp.where(z2, -1, i2)
    w1_ref[...] = jnp.broadcast_to(jnp.where(z1, 0.0, s1), (T, 16))
    w2_ref[...] = jnp.broadcast_to(jnp.where(z2, 0.0, s2), (T, 16))
    zw_ref[...] = jnp.broadcast_to(
        jnp.where(z1, s1, 0.0) + jnp.where(z2, s2, 0.0), (T, 16))


def _router(x, rw_pad, bias_pad):
    v = jax.ShapeDtypeStruct((T, 16), jnp.float32)
    iv = jax.ShapeDtypeStruct((T, 1), jnp.int32)
    return pl.pallas_call(_router_body, out_shape=(iv, iv, v, v, v))(
        x, rw_pad, bias_pad)


def _dispatch_body(e1_hbm, e2_hbm, x_hbm,
                   pos1_hbm, pos2_hbm, te_hbm, xt_hbm, na_hbm, xd_hbm,
                   e1_v, e2_v, cnt_v, hist_v, pos_v, tok_v, idx_v, zero_v,
                   src_v, meta_v, meta2_v, rows_v, hist_sh, src_sh, sem):
    wid = lax.axis_index("s")
    iota = lax.broadcasted_iota(jnp.int32, (16,), 0)
    zeros = jnp.zeros((16,), jnp.int32)

    for j in range(PADROWS // NW1 // 16):
        zero_v[pl.ds(j * 16, 16)] = zeros
    pltpu.sync_copy(zero_v, src_sh.at[pl.ds(wid * (PADROWS // NW1), PADROWS // NW1)])

    pltpu.sync_copy(e1_hbm.at[pl.ds(wid * TPW1, TPW1)], e1_v)
    pltpu.sync_copy(e2_hbm.at[pl.ds(wid * TPW1, TPW1)], e2_v)
    counts = zeros
    for src in (e1_v, e2_v):
        for j in range(TPW1 // 16):
            v = src[pl.ds(j * 16, 16)]
            for e in range(N_ROUTED):
                c = jnp.sum((v == e).astype(jnp.int32))
                counts = counts + jnp.where(iota == e, c, 0)
    cnt_v[...] = counts
    pltpu.sync_copy(cnt_v, hist_sh.at[pl.ds(wid * 16, 16)])
    plsc.subcore_barrier()

    pltpu.sync_copy(hist_sh, hist_v)
    base = zeros
    tot = zeros
    for w in range(NW1):
        row = hist_v[pl.ds(w * 16, 16)]
        tot = tot + row
        base = base + jnp.where(w < wid, row, zeros)
    padded = ((tot + (TILE - 1)) // TILE) * TILE
    offs_inc = plsc.cumsum(padded)
    offs_exc = offs_inc - padded
    start = offs_exc + base
    total_rows = jnp.max(offs_inc)
    ntact = total_rows // TILE

    running = start
    for si, src in enumerate((e1_v, e2_v)):
        for j in range(TPW1 // 16):
            v = src[pl.ds(j * 16, 16)]
            posv = zeros
            addc = zeros
            for e in range(N_ROUTED):
                mk = v == e
                mi = mk.astype(jnp.int32)
                prefix = plsc.cumsum(mi) - mi
                st_e = jnp.sum(jnp.where(iota == e, running, 0))
                posv = jnp.where(mk, st_e + prefix, posv)
                addc = addc + jnp.where(iota == e, jnp.sum(mi), 0)
            running = running + addc
            pos_v[pl.ds(si * TPW1 + j * 16, 16)] = posv
            tok_v[si, pl.ds(j * 16, 16)] = wid * TPW1 + j * 16 + iota
            idx_v[si, pl.ds(j * 16, 16)] = jnp.where(
                v >= 0, posv, PADROWS + wid)
    for si in range(2):
        pltpu.sync_copy(tok_v.at[si], src_sh.at[idx_v.at[si]])
    pltpu.sync_copy(pos_v.at[pl.ds(0, TPW1)], pos1_hbm.at[pl.ds(wid * TPW1, TPW1)])
    pltpu.sync_copy(pos_v.at[pl.ds(TPW1, TPW1)], pos2_hbm.at[pl.ds(wid * TPW1, TPW1)])

    los = []
    his = []
    for e in range(N_ROUTED):
        los.append(jnp.sum(jnp.where(iota == e, offs_exc, 0)) // TILE)
        his.append(jnp.sum(jnp.where(iota == e, offs_inc, 0)) // TILE)
    lt = ntact - 1
    last_e = jnp.int32(0)
    for e in range(N_ROUTED):
        last_e = jnp.where((lt >= los[e]) & (lt < his[e]), e, last_e)

    @pl.when(wid == 0)
    def _():
        for j in range(NTP // 16):
            tv = iota + j * 16
            texp = zeros + last_e
            for e in range(N_ROUTED):
                texp = jnp.where((tv >= los[e]) & (tv < his[e]), e, texp)
            meta_v[pl.ds(j * 16, 16)] = texp
            meta2_v[pl.ds(j * 16, 16)] = jnp.minimum(
                tv, jnp.maximum(ntact - 1, 0))
        pltpu.sync_copy(meta_v, te_hbm)
        pltpu.sync_copy(meta2_v, xt_hbm)
        cnt_v[...] = zeros + ntact
        pltpu.sync_copy(cnt_v, na_hbm)

    plsc.subcore_barrier()
    for j in range(NCHUNK // NW1):
        c = wid + j * NW1

        @pl.when(c * GCHUNK < total_rows)
        def _(c=c):
            pltpu.sync_copy(src_sh.at[pl.ds(c * GCHUNK, GCHUNK)], src_v)
            pltpu.async_copy(x_hbm.at[src_v], rows_v, sem).wait()
            pltpu.sync_copy(rows_v, xd_hbm.at[pl.ds(c * GCHUNK, GCHUNK)])


def _dispatch(e1, e2, x):
    mesh = plsc.VectorSubcoreMesh(core_axis_name="c", subcore_axis_name="s",
                                  num_cores=1, num_subcores=NW1)
    f = pl.kernel(
        _dispatch_body,
        out_type=[
            jax.ShapeDtypeStruct((T,), jnp.int32),
            jax.ShapeDtypeStruct((T,), jnp.int32),
            jax.ShapeDtypeStruct((NTP,), jnp.int32),
            jax.ShapeDtypeStruct((NTP,), jnp.int32),
            jax.ShapeDtypeStruct((16,), jnp.int32),
            jax.ShapeDtypeStruct((PADROWS, HIDDEN), jnp.float32),
        ],
        mesh=mesh,
        compiler_params=pltpu.CompilerParams(needs_layout_passes=False),
        scratch_types=[
            pltpu.VMEM((TPW1,), jnp.int32),
            pltpu.VMEM((TPW1,), jnp.int32),
            pltpu.VMEM((16,), jnp.int32),
            pltpu.VMEM((NW1 * 16,), jnp.int32),
            pltpu.VMEM((TPW1 * 2,), jnp.int32),
            pltpu.VMEM((2, TPW1), jnp.int32),
            pltpu.VMEM((2, TPW1), jnp.int32),
            pltpu.VMEM((PADROWS // NW1,), jnp.int32),
            pltpu.VMEM((GCHUNK,), jnp.int32),
            pltpu.VMEM((NTP,), jnp.int32),
            pltpu.VMEM((NTP,), jnp.int32),
            pltpu.VMEM((GCHUNK, HIDDEN), jnp.float32),
            pltpu.VMEM_SHARED((NW1 * 16,), jnp.int32),
            pltpu.VMEM_SHARED((SRCN,), jnp.int32),
            pltpu.SemaphoreType.DMA,
        ],
    )
    return f(e1, e2, x)


def _gemm_body(te_ref, xt_ref, na_ref, xd_ref, wg_ref, wu_ref, wd_ref, y_ref):
    t = pl.program_id(0)

    @pl.when(t < na_ref[0])
    def _():
        x = xd_ref[...]
        g = jnp.dot(x, wg_ref[0], preferred_element_type=jnp.float32)
        u = jnp.dot(x, wu_ref[0], preferred_element_type=jnp.float32)
        h = g * lax.logistic(g) * u
        y_ref[...] = jnp.dot(h, wd_ref[0], preferred_element_type=jnp.float32)


def _gemm(te, xt, na, xdisp, w_gate, w_up, w_down):
    grid_spec = pltpu.PrefetchScalarGridSpec(
        num_scalar_prefetch=3,
        grid=(NTILES,),
        in_specs=[
            pl.BlockSpec((TILE, HIDDEN), lambda t, te, xt, na: (xt[t], 0)),
            pl.BlockSpec((1, HIDDEN, DFF), lambda t, te, xt, na: (te[t], 0, 0)),
            pl.BlockSpec((1, HIDDEN, DFF), lambda t, te, xt, na: (te[t], 0, 0)),
            pl.BlockSpec((1, DFF, HIDDEN), lambda t, te, xt, na: (te[t], 0, 0)),
        ],
        out_specs=pl.BlockSpec((TILE, HIDDEN), lambda t, te, xt, na: (t, 0)),
    )
    return pl.pallas_call(
        _gemm_body,
        grid_spec=grid_spec,
        out_shape=jax.ShapeDtypeStruct((PADROWS, HIDDEN), jnp.float32),
    )(te, xt, na, xdisp, w_gate, w_up, w_down)


def _combine_body(x_hbm, y_hbm, p1_hbm, p2_hbm, w1_hbm, w2_hbm, zw_hbm,
                  out_hbm, p_v, w_v, x0_v, x1_v, y10_v, y11_v, y20_v, y21_v,
                  o0_v, o1_v, sem0, sem1):
    wid = lax.axis_index("s") * 2 + lax.axis_index("c")
    base = wid * TPW2
    wn = TPW2 * 16
    pltpu.sync_copy(p1_hbm.at[pl.ds(base, TPW2)], p_v.at[pl.ds(0, TPW2)])
    pltpu.sync_copy(p2_hbm.at[pl.ds(base, TPW2)], p_v.at[pl.ds(TPW2, TPW2)])
    pltpu.sync_copy(w1_hbm.at[pl.ds(base * 16, wn)], w_v.at[pl.ds(0, wn)])
    pltpu.sync_copy(w2_hbm.at[pl.ds(base * 16, wn)], w_v.at[pl.ds(wn, wn)])
    pltpu.sync_copy(zw_hbm.at[pl.ds(base * 16, wn)], w_v.at[pl.ds(2 * wn, wn)])
    xb = (x0_v, x1_v)
    y1b = (y10_v, y11_v)
    y2b = (y20_v, y21_v)
    ob = (o0_v, o1_v)
    sems = (sem0, sem1)

    def issue(c):
        b = c % 2
        t0 = base + c * 16
        return (
            pltpu.async_copy(x_hbm.at[pl.ds(t0, 16)], xb[b], sems[b]),
            pltpu.async_copy(
                y_hbm.at[p_v.at[pl.ds(c * 16, 16)]], y1b[b], sems[b]),
            pltpu.async_copy(
                y_hbm.at[p_v.at[pl.ds(TPW2 + c * 16, 16)]], y2b[b], sems[b]),
        )

    nc = TPW2 // 16
    pend = {0: issue(0)}
    for c in range(nc):
        b = c % 2
        for cp in pend.pop(c):
            cp.wait()
        if c + 1 < nc:
            pend[c + 1] = issue(c + 1)
        x_v, y1_v, y2_v, o_v = xb[b], y1b[b], y2b[b], ob[b]

        @plsc.parallel_loop(0, 16, unroll=2)
        def tok_body(l, x_v=x_v, y1_v=y1_v, y2_v=y2_v, o_v=o_v, c=c):
            w1r = w_v[pl.ds(c * 256 + l * 16, 16)]
            w2r = w_v[pl.ds(wn + c * 256 + l * 16, 16)]
            zwr = w_v[pl.ds(2 * wn + c * 256 + l * 16, 16)]
            for f in range(HIDDEN // 16):
                xr = x_v[l, pl.ds(f * 16, 16)]
                y1r = y1_v[l, pl.ds(f * 16, 16)]
                y2r = y2_v[l, pl.ds(f * 16, 16)]
                o = zwr * xr
                o = o + jnp.where(w1r != 0.0, w1r * y1r, 0.0)
                o = o + jnp.where(w2r != 0.0, w2r * y2r, 0.0)
                o_v[l, pl.ds(f * 16, 16)] = o
        pltpu.sync_copy(o_v, out_hbm.at[pl.ds(base + c * 16, 16)])


def _combine(x, y, pos1, pos2, w1, w2, zw):
    mesh = plsc.VectorSubcoreMesh(core_axis_name="c", subcore_axis_name="s",
                                  num_cores=2, num_subcores=16)
    f = pl.kernel(
        _combine_body,
        out_type=jax.ShapeDtypeStruct((T, HIDDEN), jnp.float32),
        mesh=mesh,
        compiler_params=pltpu.CompilerParams(needs_layout_passes=False),
        scratch_types=[
            pltpu.VMEM((2 * TPW2,), jnp.int32),
            pltpu.VMEM((3 * TPW2 * 16,), jnp.float32),
            pltpu.VMEM((16, HIDDEN), jnp.float32),
            pltpu.VMEM((16, HIDDEN), jnp.float32),
            pltpu.VMEM((16, HIDDEN), jnp.float32),
            pltpu.VMEM((16, HIDDEN), jnp.float32),
            pltpu.VMEM((16, HIDDEN), jnp.float32),
            pltpu.VMEM((16, HIDDEN), jnp.float32),
            pltpu.VMEM((16, HIDDEN), jnp.float32),
            pltpu.VMEM((16, HIDDEN), jnp.float32),
            pltpu.SemaphoreType.DMA,
            pltpu.SemaphoreType.DMA,
        ],
    )
    return f(x, y, pos1, pos2, w1, w2, zw)


def kernel(hidden_states, router_weight, e_score_correction_bias, w_gate, w_up, w_down):
    rw_pad = jnp.zeros((NPAD, HIDDEN), jnp.float32).at[:N_TOTAL].set(router_weight)
    bias_pad = jnp.full((1, NPAD), NEG, jnp.float32).at[0, :N_TOTAL].set(
        e_score_correction_bias)
    e1, e2, w1, w2, zw = _router(hidden_states, rw_pad, bias_pad)
    pos1, pos2, te, xt, na, xdisp = _dispatch(
        e1.reshape(T), e2.reshape(T), hidden_states)
    y = _gemm(te, xt, na, xdisp, w_gate, w_up, w_down)
    return _combine(hidden_states, y, pos1, pos2,
                    w1.reshape(T * 16), w2.reshape(T * 16), zw.reshape(T * 16))


# speedup vs baseline: 1.9396x; 1.9352x over previous
import functools

import jax
import jax.numpy as jnp
from jax import lax
from jax.experimental import pallas as pl
from jax.experimental.pallas import tpu as pltpu
from jax.experimental.pallas import tpu_sc as plsc

T = 2048
HIDDEN = 768
DFF = 512
N_ROUTED = 8
N_TOTAL = 72
NPAD = 128
SCALE = 2.5
NEG = -1e30

TILE = 128
NTILES = 40
PADROWS = NTILES * TILE
NTP = 48
NW1 = 16
TPW1 = T // NW1
NW2 = 32
TPW2 = T // NW2
SRCN = PADROWS + 64
GCHUNK = 64
NCHUNK = PADROWS // GCHUNK
LLEN = 160


def _router_body(x_ref, rw_ref, bias_ref, e1_ref, e2_ref, w1_ref, w2_ref, zw_ref):
    x = x_ref[...]
    logits = lax.dot_general(x, rw_ref[...], (((1,), (1,)), ((), ())),
                             preferred_element_type=jnp.float32)
    col = lax.broadcasted_iota(jnp.int32, (T, NPAD), 1)
    valid = col < N_TOTAL
    logits = jnp.where(valid, logits, NEG)
    m = jnp.max(logits, axis=1, keepdims=True)
    p = jnp.exp(logits - m)
    p = jnp.where(valid, p, 0.0)
    scores = p / jnp.sum(p, axis=1, keepdims=True)
    sfc = jnp.where(valid, scores + bias_ref[...], NEG)

    m1 = jnp.max(sfc, axis=1, keepdims=True)
    i1 = jnp.min(jnp.where(sfc == m1, col, NPAD), axis=1, keepdims=True)
    sfc2 = jnp.where(col == i1, NEG, sfc)
    m2 = jnp.max(sfc2, axis=1, keepdims=True)
    i2 = jnp.min(jnp.where(sfc2 == m2, col, NPAD), axis=1, keepdims=True)

    s1 = jnp.sum(jnp.where(col == i1, scores, 0.0), axis=1, keepdims=True) * SCALE
    s2 = jnp.sum(jnp.where(col == i2, scores, 0.0), axis=1, keepdims=True) * SCALE

    z1 = i1 >= N_ROUTED
    z2 = i2 >= N_ROUTED
    e1_ref[...] = jnp.where(z1, -1, i1)
    e2_ref[...] = jnp.where(z2, -1, i2)
    w1_ref[...] = jnp.where(z1, 0.0, s1)
    w2_ref[...] = jnp.where(z2, 0.0, s2)
    zw_ref[...] = jnp.broadcast_to(
        jnp.where(z1, s1, 0.0) + jnp.where(z2, s2, 0.0), (T, 16))


def _router(x, rw_pad, bias_pad):
    v = jax.ShapeDtypeStruct((T, 1), jnp.float32)
    vz = jax.ShapeDtypeStruct((T, 16), jnp.float32)
    iv = jax.ShapeDtypeStruct((T, 1), jnp.int32)
    return pl.pallas_call(_router_body, out_shape=(iv, iv, v, v, vz))(
        x, rw_pad, bias_pad)


def _dispatch_body(e1_hbm, e2_hbm, w1_hbm, w2_hbm, x_hbm,
                   te_hbm, xt_hbm, na_hbm, xd_hbm,
                   cp_hbm, cs_hbm, cw_hbm, cc_hbm,
                   e1_v, e2_v, w1_v, w2_v, cnt_v, hist_v, tok_v, idx_v,
                   zero_v, src_v, meta_v, meta2_v, rows_v,
                   lp_v, ls_v, lw_v, hist_sh, src_sh, sem):
    wid = lax.axis_index("s")
    iota = lax.broadcasted_iota(jnp.int32, (16,), 0)
    zeros = jnp.zeros((16,), jnp.int32)
    fzeros = jnp.zeros((16,), jnp.float32)

    for j in range(PADROWS // NW1 // 16):
        zero_v[pl.ds(j * 16, 16)] = zeros
    pltpu.sync_copy(zero_v, src_sh.at[pl.ds(wid * (PADROWS // NW1), PADROWS // NW1)])

    pltpu.sync_copy(e1_hbm.at[pl.ds(wid * TPW1, TPW1)], e1_v)
    pltpu.sync_copy(e2_hbm.at[pl.ds(wid * TPW1, TPW1)], e2_v)
    pltpu.sync_copy(w1_hbm.at[pl.ds(wid * TPW1, TPW1)], w1_v)
    pltpu.sync_copy(w2_hbm.at[pl.ds(wid * TPW1, TPW1)], w2_v)
    counts = zeros
    for src in (e1_v, e2_v):
        for j in range(TPW1 // 16):
            v = src[pl.ds(j * 16, 16)]
            for e in range(N_ROUTED):
                c = jnp.sum((v == e).astype(jnp.int32))
                counts = counts + jnp.where(iota == e, c, 0)
    cnt_v[...] = counts
    pltpu.sync_copy(cnt_v, hist_sh.at[pl.ds(wid * 16, 16)])
    plsc.subcore_barrier()

    pltpu.sync_copy(hist_sh, hist_v)
    base = zeros
    tot = zeros
    for w in range(NW1):
        row = hist_v[pl.ds(w * 16, 16)]
        tot = tot + row
        base = base + jnp.where(w < wid, row, zeros)
    padded = ((tot + (TILE - 1)) // TILE) * TILE
    offs_inc = plsc.cumsum(padded)
    offs_exc = offs_inc - padded
    start = offs_exc + base
    total_rows = jnp.max(offs_inc)
    ntact = total_rows // TILE

    running = start
    for h in range(2):
        for j in range(LLEN // 16):
            lp_v[pl.ds(j * 16, 16)] = zeros
            ls_v[pl.ds(j * 16, 16)] = zeros
            lw_v[pl.ds(j * 16, 16)] = fzeros
        c2 = jnp.int32(0)
        for si, (src, wsrc) in enumerate(((e1_v, w1_v), (e2_v, w2_v))):
            for j in range(TPW1 // 32):
                off = h * (TPW1 // 2) + j * 16
                v = src[pl.ds(off, 16)]
                posv = zeros
                addc = zeros
                for e in range(N_ROUTED):
                    mk = v == e
                    mi = mk.astype(jnp.int32)
                    prefix = plsc.cumsum(mi) - mi
                    st_e = jnp.sum(jnp.where(iota == e, running, 0))
                    posv = jnp.where(mk, st_e + prefix, posv)
                    addc = addc + jnp.where(iota == e, jnp.sum(mi), 0)
                running = running + addc
                tok_v[2 * h + si, pl.ds(j * 16, 16)] = wid * TPW1 + off + iota
                valid = v >= 0
                idx_v[2 * h + si, pl.ds(j * 16, 16)] = jnp.where(
                    valid, posv, PADROWS + wid)
                vi = valid.astype(jnp.int32)
                lpos = c2 + plsc.cumsum(vi) - vi
                plsc.store_scatter(lp_v, [lpos], posv, mask=valid)
                plsc.store_scatter(ls_v, [lpos], j * 16 + iota, mask=valid)
                plsc.store_scatter(lw_v, [lpos], wsrc[pl.ds(off, 16)],
                                   mask=valid)
                c2 = c2 + jnp.sum(vi)
        w2r = 2 * wid + h
        pltpu.sync_copy(lp_v, cp_hbm.at[w2r])
        pltpu.sync_copy(ls_v, cs_hbm.at[w2r])
        pltpu.sync_copy(lw_v, cw_hbm.at[w2r])
        cnt_v[...] = zeros + c2
        pltpu.sync_copy(cnt_v, cc_hbm.at[w2r])
    for q in range(4):
        pltpu.sync_copy(tok_v.at[q], src_sh.at[idx_v.at[q]])

    los = []
    his = []
    for e in range(N_ROUTED):
        los.append(jnp.sum(jnp.where(iota == e, offs_exc, 0)) // TILE)
        his.append(jnp.sum(jnp.where(iota == e, offs_inc, 0)) // TILE)
    lt = ntact - 1
    last_e = jnp.int32(0)
    for e in range(N_ROUTED):
        last_e = jnp.where((lt >= los[e]) & (lt < his[e]), e, last_e)

    @pl.when(wid == 0)
    def _():
        for j in range(NTP // 16):
            tv = iota + j * 16
            texp = zeros + last_e
            for e in range(N_ROUTED):
                texp = jnp.where((tv >= los[e]) & (tv < his[e]), e, texp)
            meta_v[pl.ds(j * 16, 16)] = texp
            meta2_v[pl.ds(j * 16, 16)] = jnp.minimum(
                tv, jnp.maximum(ntact - 1, 0))
        pltpu.sync_copy(meta_v, te_hbm)
        pltpu.sync_copy(meta2_v, xt_hbm)
        cnt_v[...] = zeros + ntact
        pltpu.sync_copy(cnt_v, na_hbm)

    plsc.subcore_barrier()
    for j in range(NCHUNK // NW1):
        c = wid + j * NW1

        @pl.when(c * GCHUNK < total_rows)
        def _(c=c):
            pltpu.sync_copy(src_sh.at[pl.ds(c * GCHUNK, GCHUNK)], src_v)
            pltpu.async_copy(x_hbm.at[src_v], rows_v, sem).wait()
            pltpu.sync_copy(rows_v, xd_hbm.at[pl.ds(c * GCHUNK, GCHUNK)])


def _dispatch(e1, e2, w1, w2, x):
    mesh = plsc.VectorSubcoreMesh(core_axis_name="c", subcore_axis_name="s",
                                  num_cores=1, num_subcores=NW1)
    f = pl.kernel(
        _dispatch_body,
        out_type=[
            jax.ShapeDtypeStruct((NTP,), jnp.int32),
            jax.ShapeDtypeStruct((NTP,), jnp.int32),
            jax.ShapeDtypeStruct((16,), jnp.int32),
            jax.ShapeDtypeStruct((PADROWS, HIDDEN), jnp.float32),
            jax.ShapeDtypeStruct((NW2, LLEN), jnp.int32),
            jax.ShapeDtypeStruct((NW2, LLEN), jnp.int32),
            jax.ShapeDtypeStruct((NW2, LLEN), jnp.float32),
            jax.ShapeDtypeStruct((NW2, 16), jnp.int32),
        ],
        mesh=mesh,
        compiler_params=pltpu.CompilerParams(needs_layout_passes=False),
        scratch_types=[
            pltpu.VMEM((TPW1,), jnp.int32),
            pltpu.VMEM((TPW1,), jnp.int32),
            pltpu.VMEM((TPW1,), jnp.float32),
            pltpu.VMEM((TPW1,), jnp.float32),
            pltpu.VMEM((16,), jnp.int32),
            pltpu.VMEM((NW1 * 16,), jnp.int32),
            pltpu.VMEM((4, TPW1 // 2), jnp.int32),
            pltpu.VMEM((4, TPW1 // 2), jnp.int32),
            pltpu.VMEM((PADROWS // NW1,), jnp.int32),
            pltpu.VMEM((GCHUNK,), jnp.int32),
            pltpu.VMEM((NTP,), jnp.int32),
            pltpu.VMEM((NTP,), jnp.int32),
            pltpu.VMEM((GCHUNK, HIDDEN), jnp.float32),
            pltpu.VMEM((LLEN,), jnp.int32),
            pltpu.VMEM((LLEN,), jnp.int32),
            pltpu.VMEM((LLEN,), jnp.float32),
            pltpu.VMEM_SHARED((NW1 * 16,), jnp.int32),
            pltpu.VMEM_SHARED((SRCN,), jnp.int32),
            pltpu.SemaphoreType.DMA,
        ],
    )
    return f(e1, e2, w1, w2, x)


def _gemm_body(te_ref, xt_ref, na_ref, xd_ref, wg_ref, wu_ref, wd_ref, y_ref):
    t = pl.program_id(0)

    @pl.when(t < na_ref[0])
    def _():
        x = xd_ref[...]
        g = jnp.dot(x, wg_ref[0], preferred_element_type=jnp.float32)
        u = jnp.dot(x, wu_ref[0], preferred_element_type=jnp.float32)
        h = g * lax.logistic(g) * u
        y_ref[...] = jnp.dot(h, wd_ref[0], preferred_element_type=jnp.float32)


def _gemm(te, xt, na, xdisp, w_gate, w_up, w_down):
    grid_spec = pltpu.PrefetchScalarGridSpec(
        num_scalar_prefetch=3,
        grid=(NTILES,),
        in_specs=[
            pl.BlockSpec((TILE, HIDDEN), lambda t, te, xt, na: (xt[t], 0)),
            pl.BlockSpec((1, HIDDEN, DFF), lambda t, te, xt, na: (te[t], 0, 0)),
            pl.BlockSpec((1, HIDDEN, DFF), lambda t, te, xt, na: (te[t], 0, 0)),
            pl.BlockSpec((1, DFF, HIDDEN), lambda t, te, xt, na: (te[t], 0, 0)),
        ],
        out_specs=pl.BlockSpec((TILE, HIDDEN), lambda t, te, xt, na: (t, 0)),
    )
    return pl.pallas_call(
        _gemm_body,
        grid_spec=grid_spec,
        out_shape=jax.ShapeDtypeStruct((PADROWS, HIDDEN), jnp.float32),
    )(te, xt, na, xdisp, w_gate, w_up, w_down)


def _combine_body(x_hbm, y_hbm, zw_hbm, cp_hbm, cs_hbm, cw_hbm, cc_hbm,
                  out_hbm, x_v, o_v, yg_v, zw_v, lp_v, ls_v, lw_v, cc_v, sem):
    wid = lax.axis_index("s") * 2 + lax.axis_index("c")
    iota = lax.broadcasted_iota(jnp.int32, (16,), 0)
    base = wid * TPW2
    pltpu.sync_copy(x_hbm.at[pl.ds(base, TPW2)], x_v)
    pltpu.sync_copy(zw_hbm.at[pl.ds(base * 16, TPW2 * 16)], zw_v)
    pltpu.sync_copy(cp_hbm.at[wid], lp_v)
    pltpu.sync_copy(cs_hbm.at[wid], ls_v)
    pltpu.sync_copy(cw_hbm.at[wid], lw_v)
    pltpu.sync_copy(cc_hbm.at[wid], cc_v)
    cnt = jnp.sum(jnp.where(iota == 0, cc_v[...], 0))

    @plsc.parallel_loop(0, TPW2, unroll=2)
    def _(t):
        zwr = zw_v[pl.ds(t * 16, 16)]
        for f in range(HIDDEN // 16):
            o_v[pl.ds(t * HIDDEN + f * 16, 16)] = zwr * x_v[t, pl.ds(f * 16, 16)]

    ng = (cnt + 15) // 16

    def gbody(g, _):
        pltpu.async_copy(y_hbm.at[lp_v.at[pl.ds(g * 16, 16)]], yg_v, sem).wait()
        wch = lw_v[pl.ds(g * 16, 16)]
        sch = ls_v[pl.ds(g * 16, 16)]

        def rbody(r, _):
            sel = iota == r
            ws = jnp.sum(jnp.where(sel, wch, 0.0))
            ss = jnp.sum(jnp.where(sel, sch, 0))
            for f in range(HIDDEN // 16):
                yr = yg_v[r, pl.ds(f * 16, 16)]
                ov = o_v[pl.ds(ss * HIDDEN + f * 16, 16)]
                o_v[pl.ds(ss * HIDDEN + f * 16, 16)] = ov + jnp.where(
                    ws != 0.0, ws * yr, 0.0)
            return 0

        lax.fori_loop(0, 16, rbody, 0)
        return 0

    lax.fori_loop(0, ng, gbody, 0)
    pltpu.sync_copy(o_v, out_hbm.at[pl.ds(base * HIDDEN, TPW2 * HIDDEN)])


def _combine(x, y, zw, cp, cs, cw, cc):
    mesh = plsc.VectorSubcoreMesh(core_axis_name="c", subcore_axis_name="s",
                                  num_cores=2, num_subcores=16)
    f = pl.kernel(
        _combine_body,
        out_type=jax.ShapeDtypeStruct((T * HIDDEN,), jnp.float32),
        mesh=mesh,
        compiler_params=pltpu.CompilerParams(needs_layout_passes=False),
        scratch_types=[
            pltpu.VMEM((TPW2, HIDDEN), jnp.float32),
            pltpu.VMEM((TPW2 * HIDDEN,), jnp.float32),
            pltpu.VMEM((16, HIDDEN), jnp.float32),
            pltpu.VMEM((TPW2 * 16,), jnp.float32),
            pltpu.VMEM((LLEN,), jnp.int32),
            pltpu.VMEM((LLEN,), jnp.int32),
            pltpu.VMEM((LLEN,), jnp.float32),
            pltpu.VMEM((16,), jnp.int32),
            pltpu.SemaphoreType.DMA,
        ],
    )
    return f(x, y, zw, cp, cs, cw, cc)


def kernel(hidden_states, router_weight, e_score_correction_bias, w_gate, w_up, w_down):
    rw_pad = jnp.zeros((NPAD, HIDDEN), jnp.float32).at[:N_TOTAL].set(router_weight)
    bias_pad = jnp.full((1, NPAD), NEG, jnp.float32).at[0, :N_TOTAL].set(
        e_score_correction_bias)
    e1, e2, w1, w2, zw = _router(hidden_states, rw_pad, bias_pad)
    te, xt, na, xdisp, cp, cs, cw, cc = _dispatch(
        e1.reshape(T), e2.reshape(T), w1.reshape(T), w2.reshape(T),
        hidden_states)
    y = _gemm(te, xt, na, xdisp, w_gate, w_up, w_down)
    out = _combine(hidden_states, y, zw.reshape(T * 16), cp, cs, cw, cc)
    return out.reshape(T, HIDDEN)


# speedup vs baseline: 2.2967x; 1.1841x over previous
import functools

import jax
import jax.numpy as jnp
from jax import lax
from jax.experimental import pallas as pl
from jax.experimental.pallas import tpu as pltpu
from jax.experimental.pallas import tpu_sc as plsc

T = 2048
HIDDEN = 768
DFF = 512
N_ROUTED = 8
N_TOTAL = 72
NPAD = 128
SCALE = 2.5
NEG = -1e30

TILE = 128
NTILES = 40
PADROWS = NTILES * TILE
NTP = 48
NW1 = 16
TPW1 = T // NW1
NW2 = 32
TPW2 = T // NW2
SRCN = PADROWS + 64
GCHUNK = 64
NCHUNK = PADROWS // GCHUNK
LLEN = 160


def _router_body(x_ref, rw_ref, bias_ref, e1_ref, e2_ref, w1_ref, w2_ref, zw_ref):
    x = x_ref[...]
    logits = lax.dot_general(x, rw_ref[...], (((1,), (1,)), ((), ())),
                             preferred_element_type=jnp.float32)
    col = lax.broadcasted_iota(jnp.int32, (T, NPAD), 1)
    valid = col < N_TOTAL
    logits = jnp.where(valid, logits, NEG)
    m = jnp.max(logits, axis=1, keepdims=True)
    p = jnp.exp(logits - m)
    p = jnp.where(valid, p, 0.0)
    scores = p / jnp.sum(p, axis=1, keepdims=True)
    sfc = jnp.where(valid, scores + bias_ref[...], NEG)

    m1 = jnp.max(sfc, axis=1, keepdims=True)
    i1 = jnp.min(jnp.where(sfc == m1, col, NPAD), axis=1, keepdims=True)
    sfc2 = jnp.where(col == i1, NEG, sfc)
    m2 = jnp.max(sfc2, axis=1, keepdims=True)
    i2 = jnp.min(jnp.where(sfc2 == m2, col, NPAD), axis=1, keepdims=True)

    s1 = jnp.sum(jnp.where(col == i1, scores, 0.0), axis=1, keepdims=True) * SCALE
    s2 = jnp.sum(jnp.where(col == i2, scores, 0.0), axis=1, keepdims=True) * SCALE

    z1 = i1 >= N_ROUTED
    z2 = i2 >= N_ROUTED
    e1_ref[...] = jnp.where(z1, -1, i1)
    e2_ref[...] = jnp.where(z2, -1, i2)
    w1_ref[...] = jnp.where(z1, 0.0, s1)
    w2_ref[...] = jnp.where(z2, 0.0, s2)
    zw_ref[...] = jnp.broadcast_to(
        jnp.where(z1, s1, 0.0) + jnp.where(z2, s2, 0.0), (T, 16))


def _router(x, rw_pad, bias_pad):
    v = jax.ShapeDtypeStruct((T, 1), jnp.float32)
    vz = jax.ShapeDtypeStruct((T, 16), jnp.float32)
    iv = jax.ShapeDtypeStruct((T, 1), jnp.int32)
    return pl.pallas_call(_router_body, out_shape=(iv, iv, v, v, vz))(
        x, rw_pad, bias_pad)


def _dispatch_body(e1_hbm, e2_hbm, w1_hbm, w2_hbm, x_hbm,
                   te_hbm, xt_hbm, na_hbm, xd_hbm,
                   cp_hbm, cs_hbm, cw_hbm, cc_hbm,
                   e1_v, e2_v, w1_v, w2_v, cnt_v, hist_v, tok_v, idx_v,
                   zero_v, src_v, meta_v, meta2_v, rows_v,
                   lp_v, ls_v, lw_v, hist_sh, src_sh, sem):
    wid = lax.axis_index("s")
    iota = lax.broadcasted_iota(jnp.int32, (16,), 0)
    zeros = jnp.zeros((16,), jnp.int32)
    fzeros = jnp.zeros((16,), jnp.float32)

    for j in range(PADROWS // NW1 // 16):
        zero_v[pl.ds(j * 16, 16)] = zeros
    pltpu.sync_copy(zero_v, src_sh.at[pl.ds(wid * (PADROWS // NW1), PADROWS // NW1)])

    pltpu.sync_copy(e1_hbm.at[pl.ds(wid * TPW1, TPW1)], e1_v)
    pltpu.sync_copy(e2_hbm.at[pl.ds(wid * TPW1, TPW1)], e2_v)
    pltpu.sync_copy(w1_hbm.at[pl.ds(wid * TPW1, TPW1)], w1_v)
    pltpu.sync_copy(w2_hbm.at[pl.ds(wid * TPW1, TPW1)], w2_v)
    counts = zeros
    for src in (e1_v, e2_v):
        for j in range(TPW1 // 16):
            v = src[pl.ds(j * 16, 16)]
            for e in range(N_ROUTED):
                c = jnp.sum((v == e).astype(jnp.int32))
                counts = counts + jnp.where(iota == e, c, 0)
    cnt_v[...] = counts
    pltpu.sync_copy(cnt_v, hist_sh.at[pl.ds(wid * 16, 16)])
    plsc.subcore_barrier()

    pltpu.sync_copy(hist_sh, hist_v)
    base = zeros
    tot = zeros
    for w in range(NW1):
        row = hist_v[pl.ds(w * 16, 16)]
        tot = tot + row
        base = base + jnp.where(w < wid, row, zeros)
    padded = ((tot + (TILE - 1)) // TILE) * TILE
    offs_inc = plsc.cumsum(padded)
    offs_exc = offs_inc - padded
    start = offs_exc + base
    total_rows = jnp.max(offs_inc)
    ntact = total_rows // TILE

    running = start
    for h in range(2):
        for j in range(LLEN // 16):
            lp_v[pl.ds(j * 16, 16)] = zeros
            ls_v[pl.ds(j * 16, 16)] = zeros
            lw_v[pl.ds(j * 16, 16)] = fzeros
        c2 = jnp.int32(0)
        for si, (src, wsrc) in enumerate(((e1_v, w1_v), (e2_v, w2_v))):
            for j in range(TPW1 // 32):
                off = h * (TPW1 // 2) + j * 16
                v = src[pl.ds(off, 16)]
                posv = zeros
                addc = zeros
                for e in range(N_ROUTED):
                    mk = v == e
                    mi = mk.astype(jnp.int32)
                    prefix = plsc.cumsum(mi) - mi
                    st_e = jnp.sum(jnp.where(iota == e, running, 0))
                    posv = jnp.where(mk, st_e + prefix, posv)
                    addc = addc + jnp.where(iota == e, jnp.sum(mi), 0)
                running = running + addc
                tok_v[2 * h + si, pl.ds(j * 16, 16)] = wid * TPW1 + off + iota
                valid = v >= 0
                idx_v[2 * h + si, pl.ds(j * 16, 16)] = jnp.where(
                    valid, posv, PADROWS + wid)
                vi = valid.astype(jnp.int32)
                lpos = c2 + plsc.cumsum(vi) - vi
                plsc.store_scatter(lp_v, [lpos], posv, mask=valid)
                plsc.store_scatter(ls_v, [lpos], j * 16 + iota, mask=valid)
                plsc.store_scatter(lw_v, [lpos], wsrc[pl.ds(off, 16)],
                                   mask=valid)
                c2 = c2 + jnp.sum(vi)
        w2r = 2 * wid + h
        pltpu.sync_copy(lp_v, cp_hbm.at[w2r])
        pltpu.sync_copy(ls_v, cs_hbm.at[w2r])
        pltpu.sync_copy(lw_v, cw_hbm.at[w2r])
        cnt_v[...] = zeros + c2
        pltpu.sync_copy(cnt_v, cc_hbm.at[w2r])
    for q in range(4):
        pltpu.sync_copy(tok_v.at[q], src_sh.at[idx_v.at[q]])

    los = []
    his = []
    for e in range(N_ROUTED):
        los.append(jnp.sum(jnp.where(iota == e, offs_exc, 0)) // TILE)
        his.append(jnp.sum(jnp.where(iota == e, offs_inc, 0)) // TILE)
    lt = ntact - 1
    last_e = jnp.int32(0)
    for e in range(N_ROUTED):
        last_e = jnp.where((lt >= los[e]) & (lt < his[e]), e, last_e)

    @pl.when(wid == 0)
    def _():
        for j in range(NTP // 16):
            tv = iota + j * 16
            texp = zeros + last_e
            for e in range(N_ROUTED):
                texp = jnp.where((tv >= los[e]) & (tv < his[e]), e, texp)
            meta_v[pl.ds(j * 16, 16)] = texp
            meta2_v[pl.ds(j * 16, 16)] = jnp.minimum(
                tv, jnp.maximum(ntact - 1, 0))
        pltpu.sync_copy(meta_v, te_hbm)
        pltpu.sync_copy(meta2_v, xt_hbm)
        cnt_v[...] = zeros + ntact
        pltpu.sync_copy(cnt_v, na_hbm)

    plsc.subcore_barrier()
    for j in range(NCHUNK // NW1):
        c = wid + j * NW1
        r0 = c * GCHUNK
        has_valid = jnp.sum(((r0 >= offs_exc) & (r0 < offs_exc + tot))
                            .astype(jnp.int32)) > 0

        @pl.when(has_valid)
        def _(c=c):
            pltpu.sync_copy(src_sh.at[pl.ds(c * GCHUNK, GCHUNK)], src_v)
            pltpu.async_copy(x_hbm.at[src_v], rows_v, sem).wait()
            pltpu.sync_copy(rows_v, xd_hbm.at[pl.ds(c * GCHUNK, GCHUNK)])


def _dispatch(e1, e2, w1, w2, x):
    mesh = plsc.VectorSubcoreMesh(core_axis_name="c", subcore_axis_name="s",
                                  num_cores=1, num_subcores=NW1)
    f = pl.kernel(
        _dispatch_body,
        out_type=[
            jax.ShapeDtypeStruct((NTP,), jnp.int32),
            jax.ShapeDtypeStruct((NTP,), jnp.int32),
            jax.ShapeDtypeStruct((16,), jnp.int32),
            jax.ShapeDtypeStruct((PADROWS, HIDDEN), jnp.float32),
            jax.ShapeDtypeStruct((NW2, LLEN), jnp.int32),
            jax.ShapeDtypeStruct((NW2, LLEN), jnp.int32),
            jax.ShapeDtypeStruct((NW2, LLEN), jnp.float32),
            jax.ShapeDtypeStruct((NW2, 16), jnp.int32),
        ],
        mesh=mesh,
        compiler_params=pltpu.CompilerParams(needs_layout_passes=False),
        scratch_types=[
            pltpu.VMEM((TPW1,), jnp.int32),
            pltpu.VMEM((TPW1,), jnp.int32),
            pltpu.VMEM((TPW1,), jnp.float32),
            pltpu.VMEM((TPW1,), jnp.float32),
            pltpu.VMEM((16,), jnp.int32),
            pltpu.VMEM((NW1 * 16,), jnp.int32),
            pltpu.VMEM((4, TPW1 // 2), jnp.int32),
            pltpu.VMEM((4, TPW1 // 2), jnp.int32),
            pltpu.VMEM((PADROWS // NW1,), jnp.int32),
            pltpu.VMEM((GCHUNK,), jnp.int32),
            pltpu.VMEM((NTP,), jnp.int32),
            pltpu.VMEM((NTP,), jnp.int32),
            pltpu.VMEM((GCHUNK, HIDDEN), jnp.float32),
            pltpu.VMEM((LLEN,), jnp.int32),
            pltpu.VMEM((LLEN,), jnp.int32),
            pltpu.VMEM((LLEN,), jnp.float32),
            pltpu.VMEM_SHARED((NW1 * 16,), jnp.int32),
            pltpu.VMEM_SHARED((SRCN,), jnp.int32),
            pltpu.SemaphoreType.DMA,
        ],
    )
    return f(e1, e2, w1, w2, x)


def _gemm_body(te_ref, xt_ref, na_ref, xd_ref, wg_ref, wu_ref, wd_ref, y_ref):
    t = pl.program_id(0)

    @pl.when(t < na_ref[0])
    def _():
        x = xd_ref[...]
        g = jnp.dot(x, wg_ref[0], preferred_element_type=jnp.float32)
        u = jnp.dot(x, wu_ref[0], preferred_element_type=jnp.float32)
        h = g * lax.logistic(g) * u
        y_ref[...] = jnp.dot(h, wd_ref[0], preferred_element_type=jnp.float32)


def _gemm(te, xt, na, xdisp, w_gate, w_up, w_down):
    grid_spec = pltpu.PrefetchScalarGridSpec(
        num_scalar_prefetch=3,
        grid=(NTILES,),
        in_specs=[
            pl.BlockSpec((TILE, HIDDEN), lambda t, te, xt, na: (xt[t], 0)),
            pl.BlockSpec((1, HIDDEN, DFF), lambda t, te, xt, na: (te[t], 0, 0)),
            pl.BlockSpec((1, HIDDEN, DFF), lambda t, te, xt, na: (te[t], 0, 0)),
            pl.BlockSpec((1, DFF, HIDDEN), lambda t, te, xt, na: (te[t], 0, 0)),
        ],
        out_specs=pl.BlockSpec(
            (TILE, HIDDEN),
            lambda t, te, xt, na: (jnp.maximum(jnp.minimum(t, na[0] - 1), 0), 0)),
    )
    return pl.pallas_call(
        _gemm_body,
        grid_spec=grid_spec,
        out_shape=jax.ShapeDtypeStruct((PADROWS, HIDDEN), jnp.float32),
    )(te, xt, na, xdisp, w_gate, w_up, w_down)


def _combine_body(x_hbm, y_hbm, zw_hbm, cp_hbm, cs_hbm, cw_hbm, cc_hbm,
                  out_hbm, x_v, o_v, yg_v, zw_v, lp_v, ls_v, lw_v, cc_v, sem):
    wid = lax.axis_index("s") * 2 + lax.axis_index("c")
    iota = lax.broadcasted_iota(jnp.int32, (16,), 0)
    base = wid * TPW2
    pltpu.sync_copy(x_hbm.at[pl.ds(base, TPW2)], x_v)
    pltpu.sync_copy(zw_hbm.at[pl.ds(base * 16, TPW2 * 16)], zw_v)
    pltpu.sync_copy(cp_hbm.at[wid], lp_v)
    pltpu.sync_copy(cs_hbm.at[wid], ls_v)
    pltpu.sync_copy(cw_hbm.at[wid], lw_v)
    pltpu.sync_copy(cc_hbm.at[wid], cc_v)
    cnt = jnp.sum(jnp.where(iota == 0, cc_v[...], 0))

    @plsc.parallel_loop(0, TPW2, unroll=2)
    def _(t):
        zwr = zw_v[pl.ds(t * 16, 16)]
        for f in range(HIDDEN // 16):
            o_v[pl.ds(t * HIDDEN + f * 16, 16)] = zwr * x_v[t, pl.ds(f * 16, 16)]

    ng = (cnt + 15) // 16

    def gbody(g, _):
        pltpu.async_copy(y_hbm.at[lp_v.at[pl.ds(g * 16, 16)]], yg_v, sem).wait()
        wch = lw_v[pl.ds(g * 16, 16)]
        sch = ls_v[pl.ds(g * 16, 16)]

        def rbody(r, _):
            sel = iota == r
            ws = jnp.sum(jnp.where(sel, wch, 0.0))
            ss = jnp.sum(jnp.where(sel, sch, 0))
            for f in range(HIDDEN // 16):
                yr = yg_v[r, pl.ds(f * 16, 16)]
                ov = o_v[pl.ds(ss * HIDDEN + f * 16, 16)]
                o_v[pl.ds(ss * HIDDEN + f * 16, 16)] = ov + jnp.where(
                    ws != 0.0, ws * yr, 0.0)
            return 0

        lax.fori_loop(0, 16, rbody, 0)
        return 0

    lax.fori_loop(0, ng, gbody, 0)
    pltpu.sync_copy(o_v, out_hbm.at[pl.ds(base * HIDDEN, TPW2 * HIDDEN)])


def _combine(x, y, zw, cp, cs, cw, cc):
    mesh = plsc.VectorSubcoreMesh(core_axis_name="c", subcore_axis_name="s",
                                  num_cores=2, num_subcores=16)
    f = pl.kernel(
        _combine_body,
        out_type=jax.ShapeDtypeStruct((T * HIDDEN,), jnp.float32),
        mesh=mesh,
        compiler_params=pltpu.CompilerParams(needs_layout_passes=False),
        scratch_types=[
            pltpu.VMEM((TPW2, HIDDEN), jnp.float32),
            pltpu.VMEM((TPW2 * HIDDEN,), jnp.float32),
            pltpu.VMEM((16, HIDDEN), jnp.float32),
            pltpu.VMEM((TPW2 * 16,), jnp.float32),
            pltpu.VMEM((LLEN,), jnp.int32),
            pltpu.VMEM((LLEN,), jnp.int32),
            pltpu.VMEM((LLEN,), jnp.float32),
            pltpu.VMEM((16,), jnp.int32),
            pltpu.SemaphoreType.DMA,
        ],
    )
    return f(x, y, zw, cp, cs, cw, cc)


def kernel(hidden_states, router_weight, e_score_correction_bias, w_gate, w_up, w_down):
    rw_pad = jnp.zeros((NPAD, HIDDEN), jnp.float32).at[:N_TOTAL].set(router_weight)
    bias_pad = jnp.full((1, NPAD), NEG, jnp.float32).at[0, :N_TOTAL].set(
        e_score_correction_bias)
    e1, e2, w1, w2, zw = _router(hidden_states, rw_pad, bias_pad)
    te, xt, na, xdisp, cp, cs, cw, cc = _dispatch(
        e1.reshape(T), e2.reshape(T), w1.reshape(T), w2.reshape(T),
        hidden_states)
    y = _gemm(te, xt, na, xdisp, w_gate, w_up, w_down)
    out = _combine(hidden_states, y, zw.reshape(T * 16), cp, cs, cw, cc)
    return out.reshape(T, HIDDEN)
